# baseline (speedup 1.0000x reference)
import jax
import jax.numpy as jnp
from jax import lax
from jax.experimental import pallas as pl
from jax.experimental.pallas import tpu as pltpu
from jax.experimental.pallas import tpu_sc as plsc

_B = 4
_S = 2048
_H = 1024
_LANES = 16

_info = plsc.get_sparse_core_info()
_NC = _info.num_cores
_NS = _info.num_subcores
_NW = _NC * _NS
_SPW = _S // _NW
_CH = 32
_NCH = _SPW // _CH


def _body(ids_hbm, word_hbm, pos_hbm, out_hbm, pos_v, rows_v, idx_v, sem):
    wid = lax.axis_index("s") * _NC + lax.axis_index("c")
    s0 = wid * _SPW

    pltpu.sync_copy(pos_hbm.at[pl.ds(s0, _SPW)], pos_v)

    for b in range(_B):
        for ch in range(_NCH):
            base = b * _S + s0 + ch * _CH
            pltpu.sync_copy(ids_hbm.at[pl.ds(base, _CH)], idx_v)
            pltpu.async_copy(word_hbm.at[idx_v], rows_v, sem).wait()

            def _row(r, carry, ch=ch):
                pr = ch * _CH + r
                for j in range(_H // _LANES):
                    sl = pl.ds(j * _LANES, _LANES)
                    plsc.addupdate(rows_v.at[r, sl], pos_v[pr, sl])
                return carry

            lax.fori_loop(0, _CH, _row, 0)
            pltpu.sync_copy(rows_v, out_hbm.at[pl.ds(base, _CH)])


def _run(ids, word_table, pos_table):
    mesh = plsc.VectorSubcoreMesh(core_axis_name="c", subcore_axis_name="s")
    kern = pl.kernel(
        _body,
        out_type=jax.ShapeDtypeStruct((_B * _S, _H), jnp.float32),
        mesh=mesh,
        scratch_types=[
            pltpu.VMEM((_SPW, _H), jnp.float32),
            pltpu.VMEM((_CH, _H), jnp.float32),
            pltpu.VMEM((_CH,), jnp.int32),
            pltpu.SemaphoreType.DMA,
        ],
    )
    return kern(ids, word_table, pos_table)


def kernel(input_ids, word_table, pos_table):
    ids = input_ids.reshape(-1).astype(jnp.int32)
    out = _run(ids, word_table, pos_table)
    return out.reshape(_B, _S, _H)


# speedup vs baseline: 1.3822x; 1.3822x over previous
import jax
import jax.numpy as jnp
from jax import lax
from jax.experimental import pallas as pl
from jax.experimental.pallas import tpu as pltpu
from jax.experimental.pallas import tpu_sc as plsc

_B = 4
_S = 2048
_H = 1024
_LANES = 16

_info = plsc.get_sparse_core_info()
_NC = _info.num_cores
_NS = _info.num_subcores
_NW = _NC * _NS
_SPW = _S // _NW
_CHP = 8
_ROWS = _B * _CHP
_NCHK = _SPW // _CHP
_NSLOT = 3


def _body(ids_hbm, word_hbm, pos_hbm, out_hbm,
          idx_v, p0, p1, p2, r0, r1, r2,
          ps0, ps1, ps2, gs0, gs1, gs2, ss0, ss1, ss2):
    pos_bufs = (p0, p1, p2)
    row_bufs = (r0, r1, r2)
    psem = (ps0, ps1, ps2)
    gsem = (gs0, gs1, gs2)
    ssem = (ss0, ss1, ss2)

    wid = lax.axis_index("s") * _NC + lax.axis_index("c")
    s0 = wid * _SPW

    pltpu.sync_copy(ids_hbm.at[wid], idx_v)

    g_h = {}
    p_h = {}
    st_h = {}

    def start_chunk(c):
        s = c % _NSLOT
        p_h[c] = pltpu.async_copy(
            pos_hbm.at[pl.ds(s0 + c * _CHP, _CHP)], pos_bufs[s], psem[s])
        g_h[c] = pltpu.async_copy(
            word_hbm.at[idx_v.at[c]], row_bufs[s], gsem[s])

    start_chunk(0)
    start_chunk(1)

    for c in range(_NCHK):
        s = c % _NSLOT
        g_h[c].wait()
        p_h[c].wait()

        rows = row_bufs[s]
        pos = pos_bufs[s]

        def _row(r, carry):
            p = r & (_CHP - 1)
            for j in range(_H // _LANES):
                sl = pl.ds(j * _LANES, _LANES)
                plsc.addupdate(rows.at[r, sl], pos[p, sl])
            return carry

        lax.fori_loop(0, _ROWS, _row, 0)

        st = []
        for b in range(_B):
            st.append(pltpu.async_copy(
                rows.at[pl.ds(b * _CHP, _CHP)],
                out_hbm.at[pl.ds(b * _S + s0 + c * _CHP, _CHP)],
                ssem[s]))
        st_h[c] = st

        if c + 2 < _NCHK:
            if c >= 1:
                for h in st_h[c - 1]:
                    h.wait()
            start_chunk(c + 2)

    for c in (_NCHK - 3, _NCHK - 2, _NCHK - 1):
        for h in st_h[c]:
            h.wait()


def _run(ids_r, word_table, pos_table):
    mesh = plsc.VectorSubcoreMesh(core_axis_name="c", subcore_axis_name="s")
    kern = pl.kernel(
        _body,
        out_type=jax.ShapeDtypeStruct((_B * _S, _H), jnp.float32),
        mesh=mesh,
        scratch_types=[
            pltpu.VMEM((_NCHK, _ROWS), jnp.int32),
            pltpu.VMEM((_CHP, _H), jnp.float32),
            pltpu.VMEM((_CHP, _H), jnp.float32),
            pltpu.VMEM((_CHP, _H), jnp.float32),
            pltpu.VMEM((_ROWS, _H), jnp.float32),
            pltpu.VMEM((_ROWS, _H), jnp.float32),
            pltpu.VMEM((_ROWS, _H), jnp.float32),
            pltpu.SemaphoreType.DMA,
            pltpu.SemaphoreType.DMA,
            pltpu.SemaphoreType.DMA,
            pltpu.SemaphoreType.DMA,
            pltpu.SemaphoreType.DMA,
            pltpu.SemaphoreType.DMA,
            pltpu.SemaphoreType.DMA,
            pltpu.SemaphoreType.DMA,
            pltpu.SemaphoreType.DMA,
        ],
    )
    return kern(ids_r, word_table, pos_table)


def kernel(input_ids, word_table, pos_table):
    ids_r = (input_ids.astype(jnp.int32)
             .reshape(_B, _NW, _NCHK, _CHP)
             .transpose(1, 2, 0, 3)
             .reshape(_NW, _NCHK, _ROWS))
    out = _run(ids_r, word_table, pos_table)
    return out.reshape(_B, _S, _H)


# speedup vs baseline: 2.2804x; 1.6499x over previous
import jax
import jax.numpy as jnp
from jax import lax
from jax.experimental import pallas as pl
from jax.experimental.pallas import tpu as pltpu
from jax.experimental.pallas import tpu_sc as plsc

_B = 4
_S = 2048
_H = 1024
_LANES = 16

_info = plsc.get_sparse_core_info()
_NC = _info.num_cores
_NS = _info.num_subcores
_NW = _NC * _NS
_SPW = _S // _NW
_CHP = 8
_ROWS = _B * _CHP
_NCHK = _SPW // _CHP
_NSLOT = 3


def _body(ids_hbm, word_hbm, pos_hbm, out_hbm,
          idx_v, p0, p1, p2, r0, r1, r2,
          ps0, ps1, ps2, gs0, gs1, gs2, ss0, ss1, ss2):
    pos_bufs = (p0, p1, p2)
    row_bufs = (r0, r1, r2)
    psem = (ps0, ps1, ps2)
    gsem = (gs0, gs1, gs2)
    ssem = (ss0, ss1, ss2)

    wid = lax.axis_index("s") * _NC + lax.axis_index("c")
    s0 = wid * _SPW

    pltpu.sync_copy(ids_hbm.at[wid], idx_v)

    g_h = {}
    p_h = {}
    st_h = {}

    def start_chunk(c):
        s = c % _NSLOT
        p_h[c] = pltpu.async_copy(
            pos_hbm.at[pl.ds(s0 + c * _CHP, _CHP)], pos_bufs[s], psem[s])
        g_h[c] = pltpu.async_copy(
            word_hbm.at[idx_v.at[c]], row_bufs[s], gsem[s])

    start_chunk(0)
    start_chunk(1)

    for c in range(_NCHK):
        s = c % _NSLOT
        g_h[c].wait()
        p_h[c].wait()

        rows = row_bufs[s]
        pos = pos_bufs[s]

        def _grp(j, carry):
            sl = pl.ds(j * _LANES, _LANES)
            for p in range(_CHP):
                v = pos[p, sl]
                for b in range(_B):
                    plsc.addupdate(rows.at[b * _CHP + p, sl], v)
            return carry

        lax.fori_loop(0, _H // _LANES, _grp, 0)

        st = []
        for b in range(_B):
            st.append(pltpu.async_copy(
                rows.at[pl.ds(b * _CHP, _CHP)],
                out_hbm.at[pl.ds(b * _S + s0 + c * _CHP, _CHP)],
                ssem[s]))
        st_h[c] = st

        if c + 2 < _NCHK:
            if c >= 1:
                for h in st_h[c - 1]:
                    h.wait()
            start_chunk(c + 2)

    for c in (_NCHK - 3, _NCHK - 2, _NCHK - 1):
        for h in st_h[c]:
            h.wait()


def _run(ids_r, word_table, pos_table):
    mesh = plsc.VectorSubcoreMesh(core_axis_name="c", subcore_axis_name="s")
    kern = pl.kernel(
        _body,
        out_type=jax.ShapeDtypeStruct((_B * _S, _H), jnp.float32),
        mesh=mesh,
        scratch_types=[
            pltpu.VMEM((_NCHK, _ROWS), jnp.int32),
            pltpu.VMEM((_CHP, _H), jnp.float32),
            pltpu.VMEM((_CHP, _H), jnp.float32),
            pltpu.VMEM((_CHP, _H), jnp.float32),
            pltpu.VMEM((_ROWS, _H), jnp.float32),
            pltpu.VMEM((_ROWS, _H), jnp.float32),
            pltpu.VMEM((_ROWS, _H), jnp.float32),
            pltpu.SemaphoreType.DMA,
            pltpu.SemaphoreType.DMA,
            pltpu.SemaphoreType.DMA,
            pltpu.SemaphoreType.DMA,
            pltpu.SemaphoreType.DMA,
            pltpu.SemaphoreType.DMA,
            pltpu.SemaphoreType.DMA,
            pltpu.SemaphoreType.DMA,
            pltpu.SemaphoreType.DMA,
        ],
    )
    return kern(ids_r, word_table, pos_table)


def kernel(input_ids, word_table, pos_table):
    ids_r = (input_ids.astype(jnp.int32)
             .reshape(_B, _NW, _NCHK, _CHP)
             .transpose(1, 2, 0, 3)
             .reshape(_NW, _NCHK, _ROWS))
    out = _run(ids_r, word_table, pos_table)
    return out.reshape(_B, _S, _H)
